# baseline (speedup 1.0000x reference)
import jax
import jax.numpy as jnp
import numpy as np
from jax.experimental import pallas as pl
from jax.experimental.pallas import tpu as pltpu

_ANCHORS = np.array(
    [[10, 13], [16, 30], [33, 23], [30, 61], [62, 45], [59, 119],
     [116, 90], [156, 198], [373, 326]], dtype=np.float32)
_NUM_CLASSES = 80
_NA = 9
_IMG_DIM = 416.0
_IGNORE_THRES = 0.5
_OBJ_SCALE = 1.0
_NOOBJ_SCALE = 100.0
_B = 16
_G = 52
_S = _G * _G
_C = _NUM_CLASSES + 5
_CH = _NA * _C
_NT = 64
_SA = _NA * _S
_STRIDE = _IMG_DIM / _G


def _sig(v):
    return jax.nn.sigmoid(v)


def _clip(p):
    return jnp.clip(p, 1e-7, 1.0 - 1e-7)


def _lane_pick(vals, lane_idx):
    out = jnp.full(lane_idx.shape, jnp.float32(vals[0]))
    for i in range(1, len(vals)):
        out = jnp.where(lane_idx == i, jnp.float32(vals[i]), out)
    return out


def _iou_row(gw_row, gh_row):
    sa_w = _ANCHORS[:, 0] / _STRIDE
    sa_h = _ANCHORS[:, 1] / _STRIDE
    it = jax.lax.broadcasted_iota(jnp.int32, (_NA, 1), 0)
    saw = jnp.zeros((_NA, 1), jnp.float32)
    sah = jnp.zeros((_NA, 1), jnp.float32)
    for i in range(_NA):
        saw = jnp.where(it == i, jnp.float32(sa_w[i]), saw)
        sah = jnp.where(it == i, jnp.float32(sa_h[i]), sah)
    inter = jnp.minimum(saw, gw_row) * jnp.minimum(sah, gh_row)
    return inter / (saw * sah + gw_row * gh_row - inter + 1e-16)


def _best_anchor_row(iou_ra):
    bn = jnp.zeros((1, _NT), jnp.float32)
    best = iou_ra[0:1, :]
    for j in range(1, _NA):
        upd = iou_ra[j:j + 1, :] > best
        bn = jnp.where(upd, jnp.float32(j), bn)
        best = jnp.maximum(best, iou_ra[j:j + 1, :])
    return bn


def _body_a(*refs):
    x_refs = refs[:_NA]
    out_ref, loss_ref = refs[_NA], refs[_NA + 1]
    c = pl.program_id(0)

    s_in = jax.lax.broadcasted_iota(jnp.int32, (1, _S), 1)
    vs = [x_refs[j][0, 0] for j in range(_NA)]

    @pl.when(c == 0)
    def _():
        gx = (s_in - (s_in // _G) * _G).astype(jnp.float32)
        for j in range(_NA):
            out_ref[0, :, _S * j:_S * (j + 1)] = (_sig(vs[j]) + gx) * _STRIDE

    @pl.when(c == 1)
    def _():
        gy = (s_in // _G).astype(jnp.float32)
        for j in range(_NA):
            out_ref[0, :, _S * j:_S * (j + 1)] = (_sig(vs[j]) + gy) * _STRIDE

    @pl.when(c == 2)
    def _():
        for j in range(_NA):
            out_ref[0, :, _S * j:_S * (j + 1)] = \
                jnp.exp(vs[j]) * jnp.float32(_ANCHORS[j, 0])

    @pl.when(c == 3)
    def _():
        for j in range(_NA):
            out_ref[0, :, _S * j:_S * (j + 1)] = \
                jnp.exp(vs[j]) * jnp.float32(_ANCHORS[j, 1])

    @pl.when(c >= 4)
    def _():
        for j in range(_NA):
            out_ref[0, :, _S * j:_S * (j + 1)] = _sig(vs[j])

    @pl.when(c == 4)
    def _():
        acc = jnp.float32(0.0)
        for j in range(_NA):
            pc = _clip(_sig(vs[j]))
            acc = acc + jnp.sum(-jnp.log(1.0 - pc))
        loss_ref[0, 0] = _NOOBJ_SCALE * acc


def _body_g(gj_ref, gi_ref, b_ref, cell_ref, out_ref):
    t = pl.program_id(0)
    cell = cell_ref[0, 0]
    b_t = b_ref[t]
    mask = (jax.lax.broadcasted_iota(jnp.int32, (_B, _CH), 0) == b_t
            ).astype(jnp.float32)
    out_ref[...] = jnp.sum(cell * mask, axis=0,
                           keepdims=True).reshape(1, 1, _CH)


def _body_b(gf_ref, tgt_ref, tgtT_ref, corr_ref):
    b_row = tgtT_ref[0:1, :]
    lab_row = tgtT_ref[1:2, :]
    cx = tgtT_ref[2:3, :] * _G
    cy = tgtT_ref[3:4, :] * _G
    gw = tgtT_ref[4:5, :] * _G
    gh = tgtT_ref[5:6, :] * _G
    gi_row = jnp.clip(jnp.floor(cx), 0.0, _G - 1.0)
    gj_row = jnp.clip(jnp.floor(cy), 0.0, _G - 1.0)
    s_row = gj_row * _G + gi_row

    b_col = tgt_ref[:, 0:1]
    lab_col = tgt_ref[:, 1:2]
    cx_col = tgt_ref[:, 2:3] * _G
    cy_col = tgt_ref[:, 3:4] * _G
    gw_col = tgt_ref[:, 4:5] * _G
    gh_col = tgt_ref[:, 5:6] * _G
    gi_col = jnp.clip(jnp.floor(cx_col), 0.0, _G - 1.0)
    gj_col = jnp.clip(jnp.floor(cy_col), 0.0, _G - 1.0)
    s_col = gj_col * _G + gi_col

    sa_w = list(_ANCHORS[:, 0] / _STRIDE)
    sa_h = list(_ANCHORS[:, 1] / _STRIDE)

    iou_ra = _iou_row(gw, gh)
    bn_row = _best_anchor_row(iou_ra)
    saw_r = jnp.zeros((1, _NA), jnp.float32)
    sah_r = jnp.zeros((1, _NA), jnp.float32)
    it9 = jax.lax.broadcasted_iota(jnp.int32, (1, _NA), 1)
    for i in range(_NA):
        saw_r = jnp.where(it9 == i, jnp.float32(sa_w[i]), saw_r)
        sah_r = jnp.where(it9 == i, jnp.float32(sa_h[i]), sah_r)
    inter_ar = jnp.minimum(saw_r, gw_col) * jnp.minimum(sah_r, gh_col)
    iou_ar = inter_ar / (saw_r * sah_r + gw_col * gh_col - inter_ar + 1e-16)
    bn_col = jnp.zeros((_NT, 1), jnp.float32)
    best_col = iou_ar[:, 0:1]
    for j in range(1, _NA):
        upd = iou_ar[:, j:j + 1] > best_col
        bn_col = jnp.where(upd, jnp.float32(j), bn_col)
        best_col = jnp.maximum(best_col, iou_ar[:, j:j + 1])

    gf = gf_ref[:, 0, :]

    g_sel = jnp.zeros((_NT, _C), jnp.float32)
    for a in range(_NA):
        g_sel = g_sel + (bn_col == a).astype(jnp.float32) \
            * gf[:, _C * a:_C * (a + 1)]
    g = g_sel.T
    g2_col = jnp.concatenate(
        [gf[:, _C * a + 4:_C * a + 5] for a in range(_NA)], axis=1)
    g2 = g2_col.T

    key_row = b_row * jnp.float32(_S) + s_row
    key_col = b_col * jnp.float32(_S) + s_col
    it0_tt = jax.lax.broadcasted_iota(jnp.int32, (_NT, _NT), 0)
    it1_tt = jax.lax.broadcasted_iota(jnp.int32, (_NT, _NT), 1)
    lower = it0_tt < it1_tt
    upper = it0_tt > it1_tt
    same_cell = key_col == key_row

    pc2 = _clip(_sig(g2))
    bce0_2 = -jnp.log(1.0 - pc2)
    noobj_sub = jnp.float32(0.0)
    for a in range(_NA):
        zer_row = (bn_row == a) | (iou_ra[a:a + 1, :] > _IGNORE_THRES)
        zer_col = (bn_col == a) | (iou_ar[:, a:a + 1] > _IGNORE_THRES)
        dup = jnp.sum((same_cell & lower & zer_col).astype(jnp.float32),
                      axis=0, keepdims=True) > 0.0
        mask = (zer_row & (~dup)).astype(jnp.float32)
        noobj_sub = noobj_sub + jnp.sum(bce0_2[a:a + 1, :] * mask)

    bn_eq = bn_col == bn_row
    lose = jnp.sum((same_cell & upper & bn_eq).astype(jnp.float32),
                   axis=0, keepdims=True) > 0.0
    win = (~lose).astype(jnp.float32)

    px = _sig(g[0:1])
    py = _sig(g[1:2])
    pw = g[2:3]
    ph = g[3:4]
    pc = _clip(_sig(g[4:5]))
    bce1_pc = -jnp.log(pc)
    saw_t = jnp.zeros((1, _NT), jnp.float32)
    sah_t = jnp.zeros((1, _NT), jnp.float32)
    for i in range(_NA):
        saw_t = jnp.where(bn_row == i, jnp.float32(sa_w[i]), saw_t)
        sah_t = jnp.where(bn_row == i, jnp.float32(sa_h[i]), sah_t)
    tx = cx - jnp.floor(cx)
    ty = cy - jnp.floor(cy)
    tw = jnp.log(gw / saw_t + 1e-16)
    th = jnp.log(gh / sah_t + 1e-16)
    sq = (px - tx) ** 2 + (py - ty) ** 2 + (pw - tw) ** 2 + (ph - th) ** 2

    pcls = _clip(_sig(g[5:85]))
    cls0 = jnp.sum(-jnp.log(1.0 - pcls), axis=0, keepdims=True)
    itc = jax.lax.broadcasted_iota(jnp.int32, (_NUM_CLASSES, _NT), 0)
    oh_lab = (itc == jnp.broadcast_to(lab_row.astype(jnp.int32),
                                      (_NUM_CLASSES, _NT))
              ).astype(jnp.float32)
    p_lab = _clip(jnp.sum(pcls * oh_lab, axis=0, keepdims=True))

    lab_eq = lab_col == lab_row
    dup_lab = jnp.sum((same_cell & lower & bn_eq & lab_eq)
                      .astype(jnp.float32), axis=0, keepdims=True) > 0.0
    first_lab = (~dup_lab).astype(jnp.float32)

    obj_add = jnp.sum(win * (sq + _OBJ_SCALE * bce1_pc + cls0))
    lab_add = jnp.sum(first_lab * (-jnp.log(p_lab) + jnp.log(1.0 - p_lab)))

    corr_ref[0, 0] = obj_add + lab_add - _NOOBJ_SCALE * noobj_sub


@jax.jit
def kernel(x, target):
    xv = x.transpose(2, 3, 0, 1)
    t1 = xv.reshape(_S, _B, _NA, _C).transpose(3, 2, 1, 0)
    tgtT = target.T

    def _mk_spec(j):
        return pl.BlockSpec((1, 1, _B, _S), lambda c, j=j: (c, j, 0, 0))

    out, loss = pl.pallas_call(
        _body_a,
        grid=(_C,),
        in_specs=[_mk_spec(j) for j in range(_NA)],
        out_specs=[
            pl.BlockSpec((1, _B, _SA), lambda c: (c, 0, 0)),
            pl.BlockSpec((1, 1), lambda c: (0, 0), memory_space=pltpu.SMEM),
        ],
        out_shape=[
            jax.ShapeDtypeStruct((_C, _B, _SA), jnp.float32),
            jax.ShapeDtypeStruct((1, 1), jnp.float32),
        ],
        compiler_params=pltpu.CompilerParams(
            dimension_semantics=("arbitrary",)),
    )(*([t1] * _NA))

    gj_idx = jnp.clip(jnp.floor(target[:, 3] * _G), 0.0,
                      _G - 1.0).astype(jnp.int32)
    gi_idx = jnp.clip(jnp.floor(target[:, 2] * _G), 0.0,
                      _G - 1.0).astype(jnp.int32)
    b_idx = target[:, 0].astype(jnp.int32)

    gfull = pl.pallas_call(
        _body_g,
        grid_spec=pltpu.PrefetchScalarGridSpec(
            num_scalar_prefetch=3,
            grid=(_NT,),
            in_specs=[
                pl.BlockSpec((1, 1, _B, _CH),
                             lambda t, gj, gi, b: (gj[t], gi[t], 0, 0)),
            ],
            out_specs=pl.BlockSpec((1, 1, _CH), lambda t, gj, gi, b: (t, 0, 0)),
        ),
        out_shape=jax.ShapeDtypeStruct((_NT, 1, _CH), jnp.float32),
        compiler_params=pltpu.CompilerParams(
            dimension_semantics=("arbitrary",)),
    )(gj_idx, gi_idx, b_idx, xv)

    corr = pl.pallas_call(
        _body_b,
        out_specs=pl.BlockSpec(memory_space=pltpu.SMEM),
        out_shape=jax.ShapeDtypeStruct((1, 1), jnp.float32),
    )(gfull, target, tgtT)

    output = out.transpose(1, 2, 0)
    total_loss = (loss + corr).reshape(())
    return output, total_loss


# speedup vs baseline: 1.0552x; 1.0552x over previous
import jax
import jax.numpy as jnp
import numpy as np
from jax.experimental import pallas as pl
from jax.experimental.pallas import tpu as pltpu

_ANCHORS = np.array(
    [[10, 13], [16, 30], [33, 23], [30, 61], [62, 45], [59, 119],
     [116, 90], [156, 198], [373, 326]], dtype=np.float32)
_NUM_CLASSES = 80
_NA = 9
_IMG_DIM = 416.0
_IGNORE_THRES = 0.5
_OBJ_SCALE = 1.0
_NOOBJ_SCALE = 100.0
_B = 16
_G = 52
_S = _G * _G
_C = _NUM_CLASSES + 5
_CH = _NA * _C
_NT = 64
_SA = _NA * _S
_STRIDE = _IMG_DIM / _G


def _sig(v):
    return jax.nn.sigmoid(v)


def _clip(p):
    return jnp.clip(p, 1e-7, 1.0 - 1e-7)


def _lane_pick(vals, lane_idx):
    out = jnp.full(lane_idx.shape, jnp.float32(vals[0]))
    for i in range(1, len(vals)):
        out = jnp.where(lane_idx == i, jnp.float32(vals[i]), out)
    return out


def _iou_row(gw_row, gh_row):
    sa_w = _ANCHORS[:, 0] / _STRIDE
    sa_h = _ANCHORS[:, 1] / _STRIDE
    it = jax.lax.broadcasted_iota(jnp.int32, (_NA, 1), 0)
    saw = jnp.zeros((_NA, 1), jnp.float32)
    sah = jnp.zeros((_NA, 1), jnp.float32)
    for i in range(_NA):
        saw = jnp.where(it == i, jnp.float32(sa_w[i]), saw)
        sah = jnp.where(it == i, jnp.float32(sa_h[i]), sah)
    inter = jnp.minimum(saw, gw_row) * jnp.minimum(sah, gh_row)
    return inter / (saw * sah + gw_row * gh_row - inter + 1e-16)


def _best_anchor_row(iou_ra):
    bn = jnp.zeros((1, _NT), jnp.float32)
    best = iou_ra[0:1, :]
    for j in range(1, _NA):
        upd = iou_ra[j:j + 1, :] > best
        bn = jnp.where(upd, jnp.float32(j), bn)
        best = jnp.maximum(best, iou_ra[j:j + 1, :])
    return bn


def _body_a(*refs):
    x_refs = refs[:_NA]
    out_ref, loss_ref = refs[_NA], refs[_NA + 1]
    c = pl.program_id(0)

    s_in = jax.lax.broadcasted_iota(jnp.int32, (1, _S), 1)
    vs = [x_refs[j][0] for j in range(_NA)]

    @pl.when(c == 0)
    def _():
        gx = (s_in - (s_in // _G) * _G).astype(jnp.float32)
        for j in range(_NA):
            out_ref[0, :, _S * j:_S * (j + 1)] = (_sig(vs[j]) + gx) * _STRIDE

    @pl.when(c == 1)
    def _():
        gy = (s_in // _G).astype(jnp.float32)
        for j in range(_NA):
            out_ref[0, :, _S * j:_S * (j + 1)] = (_sig(vs[j]) + gy) * _STRIDE

    @pl.when(c == 2)
    def _():
        for j in range(_NA):
            out_ref[0, :, _S * j:_S * (j + 1)] = \
                jnp.exp(vs[j]) * jnp.float32(_ANCHORS[j, 0])

    @pl.when(c == 3)
    def _():
        for j in range(_NA):
            out_ref[0, :, _S * j:_S * (j + 1)] = \
                jnp.exp(vs[j]) * jnp.float32(_ANCHORS[j, 1])

    @pl.when(c >= 4)
    def _():
        for j in range(_NA):
            out_ref[0, :, _S * j:_S * (j + 1)] = _sig(vs[j])

    @pl.when(c == 4)
    def _():
        acc = jnp.float32(0.0)
        for j in range(_NA):
            pc = _clip(_sig(vs[j]))
            acc = acc + jnp.sum(-jnp.log(1.0 - pc))
        loss_ref[0, 0] = _NOOBJ_SCALE * acc


def _body_g(gj_ref, gi_ref, b_ref, cell_ref, out_ref):
    t = pl.program_id(0)
    cell = cell_ref[0, 0]
    b_t = b_ref[t]
    mask = (jax.lax.broadcasted_iota(jnp.int32, (_B, _CH), 0) == b_t
            ).astype(jnp.float32)
    out_ref[...] = jnp.sum(cell * mask, axis=0,
                           keepdims=True).reshape(1, 1, _CH)


def _body_b(gf_ref, tgt_ref, tgtT_ref, corr_ref):
    b_row = tgtT_ref[0:1, :]
    lab_row = tgtT_ref[1:2, :]
    cx = tgtT_ref[2:3, :] * _G
    cy = tgtT_ref[3:4, :] * _G
    gw = tgtT_ref[4:5, :] * _G
    gh = tgtT_ref[5:6, :] * _G
    gi_row = jnp.clip(jnp.floor(cx), 0.0, _G - 1.0)
    gj_row = jnp.clip(jnp.floor(cy), 0.0, _G - 1.0)
    s_row = gj_row * _G + gi_row

    b_col = tgt_ref[:, 0:1]
    lab_col = tgt_ref[:, 1:2]
    cx_col = tgt_ref[:, 2:3] * _G
    cy_col = tgt_ref[:, 3:4] * _G
    gw_col = tgt_ref[:, 4:5] * _G
    gh_col = tgt_ref[:, 5:6] * _G
    gi_col = jnp.clip(jnp.floor(cx_col), 0.0, _G - 1.0)
    gj_col = jnp.clip(jnp.floor(cy_col), 0.0, _G - 1.0)
    s_col = gj_col * _G + gi_col

    sa_w = list(_ANCHORS[:, 0] / _STRIDE)
    sa_h = list(_ANCHORS[:, 1] / _STRIDE)

    iou_ra = _iou_row(gw, gh)
    bn_row = _best_anchor_row(iou_ra)
    saw_r = jnp.zeros((1, _NA), jnp.float32)
    sah_r = jnp.zeros((1, _NA), jnp.float32)
    it9 = jax.lax.broadcasted_iota(jnp.int32, (1, _NA), 1)
    for i in range(_NA):
        saw_r = jnp.where(it9 == i, jnp.float32(sa_w[i]), saw_r)
        sah_r = jnp.where(it9 == i, jnp.float32(sa_h[i]), sah_r)
    inter_ar = jnp.minimum(saw_r, gw_col) * jnp.minimum(sah_r, gh_col)
    iou_ar = inter_ar / (saw_r * sah_r + gw_col * gh_col - inter_ar + 1e-16)
    bn_col = jnp.zeros((_NT, 1), jnp.float32)
    best_col = iou_ar[:, 0:1]
    for j in range(1, _NA):
        upd = iou_ar[:, j:j + 1] > best_col
        bn_col = jnp.where(upd, jnp.float32(j), bn_col)
        best_col = jnp.maximum(best_col, iou_ar[:, j:j + 1])

    gf = gf_ref[:, 0, :]

    g_sel = jnp.zeros((_NT, _C), jnp.float32)
    for a in range(_NA):
        g_sel = g_sel + (bn_col == a).astype(jnp.float32) \
            * gf[:, _C * a:_C * (a + 1)]
    g = g_sel.T
    g2_col = jnp.concatenate(
        [gf[:, _C * a + 4:_C * a + 5] for a in range(_NA)], axis=1)
    g2 = g2_col.T

    key_row = b_row * jnp.float32(_S) + s_row
    key_col = b_col * jnp.float32(_S) + s_col
    it0_tt = jax.lax.broadcasted_iota(jnp.int32, (_NT, _NT), 0)
    it1_tt = jax.lax.broadcasted_iota(jnp.int32, (_NT, _NT), 1)
    lower = it0_tt < it1_tt
    upper = it0_tt > it1_tt
    same_cell = key_col == key_row

    pc2 = _clip(_sig(g2))
    bce0_2 = -jnp.log(1.0 - pc2)
    noobj_sub = jnp.float32(0.0)
    for a in range(_NA):
        zer_row = (bn_row == a) | (iou_ra[a:a + 1, :] > _IGNORE_THRES)
        zer_col = (bn_col == a) | (iou_ar[:, a:a + 1] > _IGNORE_THRES)
        dup = jnp.sum((same_cell & lower & zer_col).astype(jnp.float32),
                      axis=0, keepdims=True) > 0.0
        mask = (zer_row & (~dup)).astype(jnp.float32)
        noobj_sub = noobj_sub + jnp.sum(bce0_2[a:a + 1, :] * mask)

    bn_eq = bn_col == bn_row
    lose = jnp.sum((same_cell & upper & bn_eq).astype(jnp.float32),
                   axis=0, keepdims=True) > 0.0
    win = (~lose).astype(jnp.float32)

    px = _sig(g[0:1])
    py = _sig(g[1:2])
    pw = g[2:3]
    ph = g[3:4]
    pc = _clip(_sig(g[4:5]))
    bce1_pc = -jnp.log(pc)
    saw_t = jnp.zeros((1, _NT), jnp.float32)
    sah_t = jnp.zeros((1, _NT), jnp.float32)
    for i in range(_NA):
        saw_t = jnp.where(bn_row == i, jnp.float32(sa_w[i]), saw_t)
        sah_t = jnp.where(bn_row == i, jnp.float32(sa_h[i]), sah_t)
    tx = cx - jnp.floor(cx)
    ty = cy - jnp.floor(cy)
    tw = jnp.log(gw / saw_t + 1e-16)
    th = jnp.log(gh / sah_t + 1e-16)
    sq = (px - tx) ** 2 + (py - ty) ** 2 + (pw - tw) ** 2 + (ph - th) ** 2

    pcls = _clip(_sig(g[5:85]))
    cls0 = jnp.sum(-jnp.log(1.0 - pcls), axis=0, keepdims=True)
    itc = jax.lax.broadcasted_iota(jnp.int32, (_NUM_CLASSES, _NT), 0)
    oh_lab = (itc == jnp.broadcast_to(lab_row.astype(jnp.int32),
                                      (_NUM_CLASSES, _NT))
              ).astype(jnp.float32)
    p_lab = _clip(jnp.sum(pcls * oh_lab, axis=0, keepdims=True))

    lab_eq = lab_col == lab_row
    dup_lab = jnp.sum((same_cell & lower & bn_eq & lab_eq)
                      .astype(jnp.float32), axis=0, keepdims=True) > 0.0
    first_lab = (~dup_lab).astype(jnp.float32)

    obj_add = jnp.sum(win * (sq + _OBJ_SCALE * bce1_pc + cls0))
    lab_add = jnp.sum(first_lab * (-jnp.log(p_lab) + jnp.log(1.0 - p_lab)))

    corr_ref[0, 0] = obj_add + lab_add - _NOOBJ_SCALE * noobj_sub


@jax.jit
def kernel(x, target):
    xv = x.transpose(2, 3, 0, 1)
    tgtT = target.T

    gj_idx = jnp.clip(jnp.floor(target[:, 3] * _G), 0.0,
                      _G - 1.0).astype(jnp.int32)
    gi_idx = jnp.clip(jnp.floor(target[:, 2] * _G), 0.0,
                      _G - 1.0).astype(jnp.int32)
    b_idx = target[:, 0].astype(jnp.int32)

    gfull = pl.pallas_call(
        _body_g,
        grid_spec=pltpu.PrefetchScalarGridSpec(
            num_scalar_prefetch=3,
            grid=(_NT,),
            in_specs=[
                pl.BlockSpec((1, 1, _B, _CH),
                             lambda t, gj, gi, b: (gj[t], gi[t], 0, 0)),
            ],
            out_specs=pl.BlockSpec((1, 1, _CH), lambda t, gj, gi, b: (t, 0, 0)),
        ),
        out_shape=jax.ShapeDtypeStruct((_NT, 1, _CH), jnp.float32),
        compiler_params=pltpu.CompilerParams(
            dimension_semantics=("arbitrary",)),
    )(gj_idx, gi_idx, b_idx, xv)

    corr = pl.pallas_call(
        _body_b,
        out_specs=pl.BlockSpec(memory_space=pltpu.SMEM),
        out_shape=jax.ShapeDtypeStruct((1, 1), jnp.float32),
    )(gfull, target, tgtT)

    t1 = xv.reshape(_S, _B, _CH).transpose(2, 1, 0)

    def _mk_spec(j):
        return pl.BlockSpec((1, _B, _S), lambda c, j=j: (_C * j + c, 0, 0))

    out, loss = pl.pallas_call(
        _body_a,
        grid=(_C,),
        in_specs=[_mk_spec(j) for j in range(_NA)],
        out_specs=[
            pl.BlockSpec((1, _B, _SA), lambda c: (c, 0, 0)),
            pl.BlockSpec((1, 1), lambda c: (0, 0), memory_space=pltpu.SMEM),
        ],
        out_shape=[
            jax.ShapeDtypeStruct((_C, _B, _SA), jnp.float32),
            jax.ShapeDtypeStruct((1, 1), jnp.float32),
        ],
        compiler_params=pltpu.CompilerParams(
            dimension_semantics=("arbitrary",)),
    )(*([t1] * _NA))

    output = out.transpose(1, 2, 0)
    total_loss = (loss + corr).reshape(())
    return output, total_loss


# speedup vs baseline: 1.0556x; 1.0003x over previous
import jax
import jax.numpy as jnp
import numpy as np
from jax.experimental import pallas as pl
from jax.experimental.pallas import tpu as pltpu

_ANCHORS = np.array(
    [[10, 13], [16, 30], [33, 23], [30, 61], [62, 45], [59, 119],
     [116, 90], [156, 198], [373, 326]], dtype=np.float32)
_NUM_CLASSES = 80
_NA = 9
_IMG_DIM = 416.0
_IGNORE_THRES = 0.5
_OBJ_SCALE = 1.0
_NOOBJ_SCALE = 100.0
_B = 16
_G = 52
_S = _G * _G
_C = _NUM_CLASSES + 5
_CH = _NA * _C
_NT = 64
_SA = _NA * _S
_STRIDE = _IMG_DIM / _G


def _sig(v):
    return jax.nn.sigmoid(v)


def _clip(p):
    return jnp.clip(p, 1e-7, 1.0 - 1e-7)


def _iou_row(gw_row, gh_row):
    sa_w = _ANCHORS[:, 0] / _STRIDE
    sa_h = _ANCHORS[:, 1] / _STRIDE
    it = jax.lax.broadcasted_iota(jnp.int32, (_NA, 1), 0)
    saw = jnp.zeros((_NA, 1), jnp.float32)
    sah = jnp.zeros((_NA, 1), jnp.float32)
    for i in range(_NA):
        saw = jnp.where(it == i, jnp.float32(sa_w[i]), saw)
        sah = jnp.where(it == i, jnp.float32(sa_h[i]), sah)
    inter = jnp.minimum(saw, gw_row) * jnp.minimum(sah, gh_row)
    return inter / (saw * sah + gw_row * gh_row - inter + 1e-16)


def _best_anchor_row(iou_ra):
    bn = jnp.zeros((1, _NT), jnp.float32)
    best = iou_ra[0:1, :]
    for j in range(1, _NA):
        upd = iou_ra[j:j + 1, :] > best
        bn = jnp.where(upd, jnp.float32(j), bn)
        best = jnp.maximum(best, iou_ra[j:j + 1, :])
    return bn


def _body_a(*refs):
    x_refs = refs[:_NA]
    out_ref, loss_ref = refs[_NA], refs[_NA + 1]
    c = pl.program_id(0)

    s_in = jax.lax.broadcasted_iota(jnp.int32, (1, _S), 1)
    vs = [x_refs[j][0] for j in range(_NA)]

    @pl.when(c == 0)
    def _():
        gx = (s_in - (s_in // _G) * _G).astype(jnp.float32)
        for j in range(_NA):
            out_ref[0, :, _S * j:_S * (j + 1)] = (_sig(vs[j]) + gx) * _STRIDE

    @pl.when(c == 1)
    def _():
        gy = (s_in // _G).astype(jnp.float32)
        for j in range(_NA):
            out_ref[0, :, _S * j:_S * (j + 1)] = (_sig(vs[j]) + gy) * _STRIDE

    @pl.when(c == 2)
    def _():
        for j in range(_NA):
            out_ref[0, :, _S * j:_S * (j + 1)] = \
                jnp.exp(vs[j]) * jnp.float32(_ANCHORS[j, 0])

    @pl.when(c == 3)
    def _():
        for j in range(_NA):
            out_ref[0, :, _S * j:_S * (j + 1)] = \
                jnp.exp(vs[j]) * jnp.float32(_ANCHORS[j, 1])

    @pl.when(c >= 4)
    def _():
        for j in range(_NA):
            out_ref[0, :, _S * j:_S * (j + 1)] = _sig(vs[j])

    @pl.when(c == 4)
    def _():
        acc = jnp.float32(0.0)
        for j in range(_NA):
            pc = _clip(_sig(vs[j]))
            acc = acc + jnp.sum(-jnp.log(1.0 - pc))
        loss_ref[0, 0] = _NOOBJ_SCALE * acc


def _body_g(gj_ref, gi_ref, b_ref, cell_ref, out_ref):
    t = pl.program_id(0)
    cell = cell_ref[0, 0]
    b_t = b_ref[t]
    mask = (jax.lax.broadcasted_iota(jnp.int32, (_B, _CH), 0) == b_t
            ).astype(jnp.float32)
    out_ref[...] = jnp.sum(cell * mask, axis=0,
                           keepdims=True).reshape(1, 1, _CH)


def _body_b(gf_ref, tgt_ref, tgtT_ref, corr_ref):
    b_row = tgtT_ref[0:1, :]
    lab_row = tgtT_ref[1:2, :]
    cx = tgtT_ref[2:3, :] * _G
    cy = tgtT_ref[3:4, :] * _G
    gw = tgtT_ref[4:5, :] * _G
    gh = tgtT_ref[5:6, :] * _G
    gi_row = jnp.clip(jnp.floor(cx), 0.0, _G - 1.0)
    gj_row = jnp.clip(jnp.floor(cy), 0.0, _G - 1.0)
    s_row = gj_row * _G + gi_row

    b_col = tgt_ref[:, 0:1]
    lab_col = tgt_ref[:, 1:2]
    cx_col = tgt_ref[:, 2:3] * _G
    cy_col = tgt_ref[:, 3:4] * _G
    gw_col = tgt_ref[:, 4:5] * _G
    gh_col = tgt_ref[:, 5:6] * _G
    gi_col = jnp.clip(jnp.floor(cx_col), 0.0, _G - 1.0)
    gj_col = jnp.clip(jnp.floor(cy_col), 0.0, _G - 1.0)
    s_col = gj_col * _G + gi_col

    sa_w = list(_ANCHORS[:, 0] / _STRIDE)
    sa_h = list(_ANCHORS[:, 1] / _STRIDE)

    iou_ra = _iou_row(gw, gh)
    bn_row = _best_anchor_row(iou_ra)
    saw_r = jnp.zeros((1, _NA), jnp.float32)
    sah_r = jnp.zeros((1, _NA), jnp.float32)
    it9 = jax.lax.broadcasted_iota(jnp.int32, (1, _NA), 1)
    for i in range(_NA):
        saw_r = jnp.where(it9 == i, jnp.float32(sa_w[i]), saw_r)
        sah_r = jnp.where(it9 == i, jnp.float32(sa_h[i]), sah_r)
    inter_ar = jnp.minimum(saw_r, gw_col) * jnp.minimum(sah_r, gh_col)
    iou_ar = inter_ar / (saw_r * sah_r + gw_col * gh_col - inter_ar + 1e-16)
    bn_col = jnp.zeros((_NT, 1), jnp.float32)
    best_col = iou_ar[:, 0:1]
    for j in range(1, _NA):
        upd = iou_ar[:, j:j + 1] > best_col
        bn_col = jnp.where(upd, jnp.float32(j), bn_col)
        best_col = jnp.maximum(best_col, iou_ar[:, j:j + 1])

    gf = gf_ref[:, 0, :]

    g_sel = jnp.zeros((_NT, _C), jnp.float32)
    for a in range(_NA):
        g_sel = g_sel + (bn_col == a).astype(jnp.float32) \
            * gf[:, _C * a:_C * (a + 1)]
    g = g_sel.T
    g2_col = jnp.concatenate(
        [gf[:, _C * a + 4:_C * a + 5] for a in range(_NA)], axis=1)
    g2 = g2_col.T

    key_row = b_row * jnp.float32(_S) + s_row
    key_col = b_col * jnp.float32(_S) + s_col
    it0_tt = jax.lax.broadcasted_iota(jnp.int32, (_NT, _NT), 0)
    it1_tt = jax.lax.broadcasted_iota(jnp.int32, (_NT, _NT), 1)
    lower = it0_tt < it1_tt
    upper = it0_tt > it1_tt
    same_cell = key_col == key_row

    pc2 = _clip(_sig(g2))
    bce0_2 = -jnp.log(1.0 - pc2)
    noobj_sub = jnp.float32(0.0)
    for a in range(_NA):
        zer_row = (bn_row == a) | (iou_ra[a:a + 1, :] > _IGNORE_THRES)
        zer_col = (bn_col == a) | (iou_ar[:, a:a + 1] > _IGNORE_THRES)
        dup = jnp.sum((same_cell & lower & zer_col).astype(jnp.float32),
                      axis=0, keepdims=True) > 0.0
        mask = (zer_row & (~dup)).astype(jnp.float32)
        noobj_sub = noobj_sub + jnp.sum(bce0_2[a:a + 1, :] * mask)

    bn_eq = bn_col == bn_row
    lose = jnp.sum((same_cell & upper & bn_eq).astype(jnp.float32),
                   axis=0, keepdims=True) > 0.0
    win = (~lose).astype(jnp.float32)

    px = _sig(g[0:1])
    py = _sig(g[1:2])
    pw = g[2:3]
    ph = g[3:4]
    pc = _clip(_sig(g[4:5]))
    bce1_pc = -jnp.log(pc)
    saw_t = jnp.zeros((1, _NT), jnp.float32)
    sah_t = jnp.zeros((1, _NT), jnp.float32)
    for i in range(_NA):
        saw_t = jnp.where(bn_row == i, jnp.float32(sa_w[i]), saw_t)
        sah_t = jnp.where(bn_row == i, jnp.float32(sa_h[i]), sah_t)
    tx = cx - jnp.floor(cx)
    ty = cy - jnp.floor(cy)
    tw = jnp.log(gw / saw_t + 1e-16)
    th = jnp.log(gh / sah_t + 1e-16)
    sq = (px - tx) ** 2 + (py - ty) ** 2 + (pw - tw) ** 2 + (ph - th) ** 2

    pcls = _clip(_sig(g[5:85]))
    cls0 = jnp.sum(-jnp.log(1.0 - pcls), axis=0, keepdims=True)
    itc = jax.lax.broadcasted_iota(jnp.int32, (_NUM_CLASSES, _NT), 0)
    oh_lab = (itc == jnp.broadcast_to(lab_row.astype(jnp.int32),
                                      (_NUM_CLASSES, _NT))
              ).astype(jnp.float32)
    p_lab = _clip(jnp.sum(pcls * oh_lab, axis=0, keepdims=True))

    lab_eq = lab_col == lab_row
    dup_lab = jnp.sum((same_cell & lower & bn_eq & lab_eq)
                      .astype(jnp.float32), axis=0, keepdims=True) > 0.0
    first_lab = (~dup_lab).astype(jnp.float32)

    obj_add = jnp.sum(win * (sq + _OBJ_SCALE * bce1_pc + cls0))
    lab_add = jnp.sum(first_lab * (-jnp.log(p_lab) + jnp.log(1.0 - p_lab)))

    corr_ref[0, 0] = obj_add + lab_add - _NOOBJ_SCALE * noobj_sub


@jax.jit
def kernel(x, target):
    xv = x.transpose(2, 3, 0, 1)
    tgtT = target.T

    gj_idx = jnp.clip(jnp.floor(target[:, 3] * _G), 0.0,
                      _G - 1.0).astype(jnp.int32)
    gi_idx = jnp.clip(jnp.floor(target[:, 2] * _G), 0.0,
                      _G - 1.0).astype(jnp.int32)
    b_idx = target[:, 0].astype(jnp.int32)

    gfull = pl.pallas_call(
        _body_g,
        grid_spec=pltpu.PrefetchScalarGridSpec(
            num_scalar_prefetch=3,
            grid=(_NT,),
            in_specs=[
                pl.BlockSpec((1, 1, _B, _CH),
                             lambda t, gj, gi, b: (gj[t], gi[t], 0, 0)),
            ],
            out_specs=pl.BlockSpec((1, 1, _CH), lambda t, gj, gi, b: (t, 0, 0)),
        ),
        out_shape=jax.ShapeDtypeStruct((_NT, 1, _CH), jnp.float32),
        compiler_params=pltpu.CompilerParams(
            dimension_semantics=("arbitrary",)),
    )(gj_idx, gi_idx, b_idx, xv)

    corr = pl.pallas_call(
        _body_b,
        out_specs=pl.BlockSpec(memory_space=pltpu.SMEM),
        out_shape=jax.ShapeDtypeStruct((1, 1), jnp.float32),
    )(gfull, target, tgtT)

    t1 = xv.reshape(_S, _B, _CH).transpose(2, 1, 0)

    def _mk_spec(j):
        return pl.BlockSpec((1, _B, _S), lambda c, j=j: (_C * j + c, 0, 0))

    out, loss = pl.pallas_call(
        _body_a,
        grid=(_C,),
        in_specs=[_mk_spec(j) for j in range(_NA)],
        out_specs=[
            pl.BlockSpec((1, _B, _SA), lambda c: (c, 0, 0)),
            pl.BlockSpec((1, 1), lambda c: (0, 0), memory_space=pltpu.SMEM),
        ],
        out_shape=[
            jax.ShapeDtypeStruct((_C, _B, _SA), jnp.float32),
            jax.ShapeDtypeStruct((1, 1), jnp.float32),
        ],
        compiler_params=pltpu.CompilerParams(
            dimension_semantics=("arbitrary",)),
    )(*([t1] * _NA))

    output = out.transpose(1, 2, 0)
    total_loss = (loss + corr).reshape(())
    return output, total_loss


# speedup vs baseline: 1.0559x; 1.0003x over previous
import jax
import jax.numpy as jnp
import numpy as np
from jax.experimental import pallas as pl
from jax.experimental.pallas import tpu as pltpu

_ANCHORS = np.array(
    [[10, 13], [16, 30], [33, 23], [30, 61], [62, 45], [59, 119],
     [116, 90], [156, 198], [373, 326]], dtype=np.float32)
_NUM_CLASSES = 80
_NA = 9
_IMG_DIM = 416.0
_IGNORE_THRES = 0.5
_OBJ_SCALE = 1.0
_NOOBJ_SCALE = 100.0
_B = 16
_G = 52
_S = _G * _G
_C = _NUM_CLASSES + 5
_CH = _NA * _C
_NT = 64
_SA = _NA * _S
_STRIDE = _IMG_DIM / _G


def _sig(v):
    return jax.nn.sigmoid(v)


def _clip(p):
    return jnp.clip(p, 1e-7, 1.0 - 1e-7)


def _iou_row(gw_row, gh_row):
    sa_w = _ANCHORS[:, 0] / _STRIDE
    sa_h = _ANCHORS[:, 1] / _STRIDE
    it = jax.lax.broadcasted_iota(jnp.int32, (_NA, 1), 0)
    saw = jnp.zeros((_NA, 1), jnp.float32)
    sah = jnp.zeros((_NA, 1), jnp.float32)
    for i in range(_NA):
        saw = jnp.where(it == i, jnp.float32(sa_w[i]), saw)
        sah = jnp.where(it == i, jnp.float32(sa_h[i]), sah)
    inter = jnp.minimum(saw, gw_row) * jnp.minimum(sah, gh_row)
    return inter / (saw * sah + gw_row * gh_row - inter + 1e-16)


def _best_anchor_row(iou_ra):
    bn = jnp.zeros((1, _NT), jnp.float32)
    best = iou_ra[0:1, :]
    for j in range(1, _NA):
        upd = iou_ra[j:j + 1, :] > best
        bn = jnp.where(upd, jnp.float32(j), bn)
        best = jnp.maximum(best, iou_ra[j:j + 1, :])
    return bn


def _body_a(*refs):
    x_refs = refs[:_NA]
    out_ref, loss_ref = refs[_NA], refs[_NA + 1]
    c = pl.program_id(0)

    s_in = jax.lax.broadcasted_iota(jnp.int32, (1, _S), 1)
    vs = [x_refs[j][0] for j in range(_NA)]

    @pl.when(c == 0)
    def _():
        gx = (s_in - (s_in // _G) * _G).astype(jnp.float32)
        for j in range(_NA):
            out_ref[0, :, _S * j:_S * (j + 1)] = (_sig(vs[j]) + gx) * _STRIDE

    @pl.when(c == 1)
    def _():
        gy = (s_in // _G).astype(jnp.float32)
        for j in range(_NA):
            out_ref[0, :, _S * j:_S * (j + 1)] = (_sig(vs[j]) + gy) * _STRIDE

    @pl.when(c == 2)
    def _():
        for j in range(_NA):
            out_ref[0, :, _S * j:_S * (j + 1)] = \
                jnp.exp(vs[j]) * jnp.float32(_ANCHORS[j, 0])

    @pl.when(c == 3)
    def _():
        for j in range(_NA):
            out_ref[0, :, _S * j:_S * (j + 1)] = \
                jnp.exp(vs[j]) * jnp.float32(_ANCHORS[j, 1])

    @pl.when(c >= 4)
    def _():
        for j in range(_NA):
            out_ref[0, :, _S * j:_S * (j + 1)] = _sig(vs[j])

    @pl.when(c == 4)
    def _():
        acc = jnp.float32(0.0)
        for j in range(_NA):
            pc = _clip(_sig(vs[j]))
            acc = acc + jnp.sum(-jnp.log(1.0 - pc))
        loss_ref[0, 0] = _NOOBJ_SCALE * acc


def _body_g(gj_ref, gi_ref, b_ref, cell_ref, out_ref):
    t = pl.program_id(0)
    cell = cell_ref[0, 0]
    b_t = b_ref[t]
    mask = (jax.lax.broadcasted_iota(jnp.int32, (_B, _CH), 0) == b_t
            ).astype(jnp.float32)
    out_ref[...] = jnp.sum(cell * mask, axis=0,
                           keepdims=True).reshape(1, 1, _CH)


def _body_b(gf_ref, tgt_ref, tgtT_ref, corr_ref):
    b_row = tgtT_ref[0:1, :]
    lab_row = tgtT_ref[1:2, :]
    cx = tgtT_ref[2:3, :] * _G
    cy = tgtT_ref[3:4, :] * _G
    gw = tgtT_ref[4:5, :] * _G
    gh = tgtT_ref[5:6, :] * _G
    gi_row = jnp.clip(jnp.floor(cx), 0.0, _G - 1.0)
    gj_row = jnp.clip(jnp.floor(cy), 0.0, _G - 1.0)
    s_row = gj_row * _G + gi_row

    b_col = tgt_ref[:, 0:1]
    lab_col = tgt_ref[:, 1:2]
    cx_col = tgt_ref[:, 2:3] * _G
    cy_col = tgt_ref[:, 3:4] * _G
    gw_col = tgt_ref[:, 4:5] * _G
    gh_col = tgt_ref[:, 5:6] * _G
    gi_col = jnp.clip(jnp.floor(cx_col), 0.0, _G - 1.0)
    gj_col = jnp.clip(jnp.floor(cy_col), 0.0, _G - 1.0)
    s_col = gj_col * _G + gi_col

    sa_w = list(_ANCHORS[:, 0] / _STRIDE)
    sa_h = list(_ANCHORS[:, 1] / _STRIDE)

    iou_ra = _iou_row(gw, gh)
    bn_row = _best_anchor_row(iou_ra)
    saw_r = jnp.zeros((1, _NA), jnp.float32)
    sah_r = jnp.zeros((1, _NA), jnp.float32)
    it9 = jax.lax.broadcasted_iota(jnp.int32, (1, _NA), 1)
    for i in range(_NA):
        saw_r = jnp.where(it9 == i, jnp.float32(sa_w[i]), saw_r)
        sah_r = jnp.where(it9 == i, jnp.float32(sa_h[i]), sah_r)
    inter_ar = jnp.minimum(saw_r, gw_col) * jnp.minimum(sah_r, gh_col)
    iou_ar = inter_ar / (saw_r * sah_r + gw_col * gh_col - inter_ar + 1e-16)
    bn_col = jnp.zeros((_NT, 1), jnp.float32)
    best_col = iou_ar[:, 0:1]
    for j in range(1, _NA):
        upd = iou_ar[:, j:j + 1] > best_col
        bn_col = jnp.where(upd, jnp.float32(j), bn_col)
        best_col = jnp.maximum(best_col, iou_ar[:, j:j + 1])

    gf = gf_ref[:, 0, :]

    g_sel = jnp.zeros((_NT, _C), jnp.float32)
    for a in range(_NA):
        g_sel = g_sel + (bn_col == a).astype(jnp.float32) \
            * gf[:, _C * a:_C * (a + 1)]
    g = g_sel.T
    g2_col = jnp.concatenate(
        [gf[:, _C * a + 4:_C * a + 5] for a in range(_NA)], axis=1)
    g2 = g2_col.T

    key_row = b_row * jnp.float32(_S) + s_row
    key_col = b_col * jnp.float32(_S) + s_col
    it0_tt = jax.lax.broadcasted_iota(jnp.int32, (_NT, _NT), 0)
    it1_tt = jax.lax.broadcasted_iota(jnp.int32, (_NT, _NT), 1)
    lower = it0_tt < it1_tt
    upper = it0_tt > it1_tt
    same_cell = key_col == key_row

    pc2 = _clip(_sig(g2))
    bce0_2 = -jnp.log(1.0 - pc2)
    noobj_sub = jnp.float32(0.0)
    for a in range(_NA):
        zer_row = (bn_row == a) | (iou_ra[a:a + 1, :] > _IGNORE_THRES)
        zer_col = (bn_col == a) | (iou_ar[:, a:a + 1] > _IGNORE_THRES)
        dup = jnp.sum((same_cell & lower & zer_col).astype(jnp.float32),
                      axis=0, keepdims=True) > 0.0
        mask = (zer_row & (~dup)).astype(jnp.float32)
        noobj_sub = noobj_sub + jnp.sum(bce0_2[a:a + 1, :] * mask)

    bn_eq = bn_col == bn_row
    lose = jnp.sum((same_cell & upper & bn_eq).astype(jnp.float32),
                   axis=0, keepdims=True) > 0.0
    win = (~lose).astype(jnp.float32)

    px = _sig(g[0:1])
    py = _sig(g[1:2])
    pw = g[2:3]
    ph = g[3:4]
    pc = _clip(_sig(g[4:5]))
    bce1_pc = -jnp.log(pc)
    saw_t = jnp.zeros((1, _NT), jnp.float32)
    sah_t = jnp.zeros((1, _NT), jnp.float32)
    for i in range(_NA):
        saw_t = jnp.where(bn_row == i, jnp.float32(sa_w[i]), saw_t)
        sah_t = jnp.where(bn_row == i, jnp.float32(sa_h[i]), sah_t)
    tx = cx - jnp.floor(cx)
    ty = cy - jnp.floor(cy)
    tw = jnp.log(gw / saw_t + 1e-16)
    th = jnp.log(gh / sah_t + 1e-16)
    sq = (px - tx) ** 2 + (py - ty) ** 2 + (pw - tw) ** 2 + (ph - th) ** 2

    pcls = _clip(_sig(g[5:85]))
    cls0 = jnp.sum(-jnp.log(1.0 - pcls), axis=0, keepdims=True)
    itc = jax.lax.broadcasted_iota(jnp.int32, (_NUM_CLASSES, _NT), 0)
    oh_lab = (itc == jnp.broadcast_to(lab_row.astype(jnp.int32),
                                      (_NUM_CLASSES, _NT))
              ).astype(jnp.float32)
    p_lab = _clip(jnp.sum(pcls * oh_lab, axis=0, keepdims=True))

    lab_eq = lab_col == lab_row
    dup_lab = jnp.sum((same_cell & lower & bn_eq & lab_eq)
                      .astype(jnp.float32), axis=0, keepdims=True) > 0.0
    first_lab = (~dup_lab).astype(jnp.float32)

    obj_add = jnp.sum(win * (sq + _OBJ_SCALE * bce1_pc + cls0))
    lab_add = jnp.sum(first_lab * (-jnp.log(p_lab) + jnp.log(1.0 - p_lab)))

    corr_ref[0, 0] = obj_add + lab_add - _NOOBJ_SCALE * noobj_sub


@jax.jit
def kernel(x, target):
    xv = x.transpose(2, 3, 0, 1)
    tgtT = target.T

    gj_idx = jnp.clip(jnp.floor(target[:, 3] * _G), 0.0,
                      _G - 1.0).astype(jnp.int32)
    gi_idx = jnp.clip(jnp.floor(target[:, 2] * _G), 0.0,
                      _G - 1.0).astype(jnp.int32)
    b_idx = target[:, 0].astype(jnp.int32)

    gfull = pl.pallas_call(
        _body_g,
        grid_spec=pltpu.PrefetchScalarGridSpec(
            num_scalar_prefetch=3,
            grid=(_NT,),
            in_specs=[
                pl.BlockSpec((1, 1, _B, _CH),
                             lambda t, gj, gi, b: (gj[t], gi[t], 0, 0)),
            ],
            out_specs=pl.BlockSpec((1, 1, _CH), lambda t, gj, gi, b: (t, 0, 0)),
        ),
        out_shape=jax.ShapeDtypeStruct((_NT, 1, _CH), jnp.float32),
        compiler_params=pltpu.CompilerParams(
            dimension_semantics=("arbitrary",)),
    )(gj_idx, gi_idx, b_idx, xv)

    corr = pl.pallas_call(
        _body_b,
        out_specs=pl.BlockSpec(memory_space=pltpu.SMEM),
        out_shape=jax.ShapeDtypeStruct((1, 1), jnp.float32),
    )(gfull, target, tgtT)

    t_mid = jax.lax.optimization_barrier(
        xv.reshape(_S, _B, _CH).transpose(1, 0, 2))
    t1 = t_mid.transpose(2, 0, 1)

    def _mk_spec(j):
        return pl.BlockSpec((1, _B, _S), lambda c, j=j: (_C * j + c, 0, 0))

    out, loss = pl.pallas_call(
        _body_a,
        grid=(_C,),
        in_specs=[_mk_spec(j) for j in range(_NA)],
        out_specs=[
            pl.BlockSpec((1, _B, _SA), lambda c: (c, 0, 0)),
            pl.BlockSpec((1, 1), lambda c: (0, 0), memory_space=pltpu.SMEM),
        ],
        out_shape=[
            jax.ShapeDtypeStruct((_C, _B, _SA), jnp.float32),
            jax.ShapeDtypeStruct((1, 1), jnp.float32),
        ],
        compiler_params=pltpu.CompilerParams(
            dimension_semantics=("arbitrary",)),
    )(*([t1] * _NA))

    output = out.transpose(1, 2, 0)
    total_loss = (loss + corr).reshape(())
    return output, total_loss
